# baseline (speedup 1.0000x reference)
import jax
import jax.numpy as jnp
from jax import lax
from jax.experimental import pallas as pl
from jax.experimental.pallas import tpu as pltpu
from jax.experimental.pallas import tpu_sc as plsc

VOCAB = 1000000
DIM = 64
B = 16384
P = 20
NPAIR = 3 * P
NW = 32
B_PER_W = B // NW
GB = 8
ROWS_G = GB * NPAIR
NG = B_PER_W // GB
OUT_W = B_PER_W * NPAIR


def _sc_body(u_idx_hbm, ctx_idx_hbm, u_emb_hbm, v_emb_hbm, y_hbm,
             u_idx_v, ctx_idx_v, u_sup_v, sup_v, rows_v, ugrp_v, tp_v, out_v,
             sem):
    wid = lax.axis_index("s") * 2 + lax.axis_index("c")
    base_b = wid * B_PER_W

    pltpu.sync_copy(u_idx_hbm.at[pl.ds(base_b, B_PER_W)],
                    u_idx_v.at[pl.ds(0, B_PER_W)])
    pltpu.sync_copy(ctx_idx_hbm.at[pl.ds(base_b * NPAIR, OUT_W)], ctx_idx_v)

    for t in range(B_PER_W // 16):
        u_sup_v[pl.ds(16 * t, 16)] = lax.shift_right_logical(
            u_idx_v[pl.ds(16 * t, 16)], 1)

    row_iota = lax.iota(jnp.int32, 16)
    cols = [jnp.full((16,), c, jnp.int32) for c in range(16)]
    ch_off = [row_iota + (16 * k) for k in range(4)]

    @pl.loop(0, NG)
    def _group(g):
        for t in range(ROWS_G // 16):
            sup_v[pl.ds(16 * t, 16)] = lax.shift_right_logical(
                ctx_idx_v[pl.ds(g * ROWS_G + 16 * t, 16)], 1)

        cps = [pltpu.async_copy(
            u_emb_hbm.at[u_sup_v.at[pl.ds(g * GB, GB)]], ugrp_v, sem)]
        for q in range(4):
            cps.append(pltpu.async_copy(
                v_emb_hbm.at[sup_v.at[pl.ds(q * 120, 120)]],
                rows_v.at[pl.ds(q * 120, 120)], sem))
        for cp in cps:
            cp.wait()

        gbase = g * ROWS_G
        uoff = (u_idx_v[pl.ds(8 * g, 16)] & 1) * 64
        u_cache = {}
        hvec = None
        for j in range(ROWS_G):
            bb, jj = divmod(j, NPAIR)
            if bb not in u_cache:
                uh = uoff[bb]
                uc = [ugrp_v[bb, pl.ds(uh + 16 * k, 16)] for k in range(4)]
                u_cache[bb] = (uc, [-c for c in uc])
            uc, nuc = u_cache[bb]
            ch = nuc if jj < P else uc
            if j % 16 == 0:
                hvec = (ctx_idx_v[pl.ds(gbase + j, 16)] & 1) * 64
            rh = hvec[j % 16]
            part = rows_v[j, pl.ds(rh, 16)] * ch[0]
            for k in range(1, 4):
                part = part + rows_v[j, pl.ds(rh + 16 * k, 16)] * ch[k]
            tp_v[j % 16] = part
            if j % 16 == 15:
                acc = plsc.load_gather(tp_v, [row_iota, cols[0]])
                for c in range(1, 16):
                    acc = acc + plsc.load_gather(tp_v, [row_iota, cols[c]])
                out_v[pl.ds(gbase + (j - 15), 16)] = acc

    pltpu.sync_copy(out_v, y_hbm.at[pl.ds(wid * OUT_W, OUT_W)])


def _tc_finish(y2d):
    nrows = y2d.shape[0]
    blk = 256
    grid = nrows // blk

    def body(y_ref, o_ref):
        i = pl.program_id(0)
        x = y_ref[...]
        sp = jnp.maximum(x, 0.0) + jnp.log1p(jnp.exp(-jnp.abs(x)))
        s = jnp.sum(sp) * (1.0 / B)

        @pl.when(i == 0)
        def _init():
            o_ref[...] = jnp.zeros_like(o_ref)

        o_ref[...] = o_ref[...] + s

    return pl.pallas_call(
        body,
        grid=(grid,),
        in_specs=[pl.BlockSpec((blk, 128), lambda i: (i, 0))],
        out_specs=pl.BlockSpec((1, 1), lambda i: (0, 0)),
        out_shape=jax.ShapeDtypeStruct((1, 1), jnp.float32),
    )(y2d)


def kernel(u, v, neg, wn, sim, not_sim, mismatch, u_emb, v_emb):
    del sim, not_sim, mismatch
    u_i = u.astype(jnp.int32)
    ctx = jnp.concatenate([v, neg, wn], axis=1).astype(jnp.int32).reshape(-1)
    ut = u_emb.reshape(VOCAB // 2, 2 * DIM)
    vt = v_emb.reshape(VOCAB // 2, 2 * DIM)

    mesh = plsc.VectorSubcoreMesh(core_axis_name="c", subcore_axis_name="s")
    y = pl.kernel(
        _sc_body,
        out_type=jax.ShapeDtypeStruct((B * NPAIR,), jnp.float32),
        mesh=mesh,
        compiler_params=pltpu.CompilerParams(needs_layout_passes=False),
        scratch_types=[
            pltpu.VMEM((B_PER_W + 16,), jnp.int32),
            pltpu.VMEM((OUT_W,), jnp.int32),
            pltpu.VMEM((B_PER_W,), jnp.int32),
            pltpu.VMEM((ROWS_G,), jnp.int32),
            pltpu.VMEM((ROWS_G, 2 * DIM), jnp.float32),
            pltpu.VMEM((GB, 2 * DIM), jnp.float32),
            pltpu.VMEM((16, 16), jnp.float32),
            pltpu.VMEM((OUT_W,), jnp.float32),
            pltpu.SemaphoreType.DMA,
        ],
    )(u_i, ctx, ut, vt)

    loss = _tc_finish(y.reshape(B * NPAIR // 128, 128))
    return loss[0, 0]


# speedup vs baseline: 1.6905x; 1.6905x over previous
import jax
import jax.numpy as jnp
from jax import lax
from jax.experimental import pallas as pl
from jax.experimental.pallas import tpu as pltpu
from jax.experimental.pallas import tpu_sc as plsc

VOCAB = 1000000
DIM = 64
B = 16384
P = 20
NPAIR = 3 * P
NW = 32
B_PER_W = B // NW
GB = 4
ROWS_G = GB * NPAIR
NG = B_PER_W // GB
OUT_W = B_PER_W * NPAIR
GRP_BYTES = ROWS_G * DIM * 4 + GB * DIM * 4


def _sc_body(ctx_idx_hbm, emb_u_hbm, v_emb_hbm, y_hbm,
             ctx_idx_v, rows0_v, rows1_v, ugrp0_v, ugrp1_v, tp_v, out_v,
             sem0, sem1):
    wid = lax.axis_index("s") * 2 + lax.axis_index("c")
    base_b = wid * B_PER_W

    pltpu.sync_copy(ctx_idx_hbm.at[pl.ds(base_b * NPAIR, OUT_W)], ctx_idx_v)

    rows = (rows0_v, rows1_v)
    ugrp = (ugrp0_v, ugrp1_v)
    sems = (sem0, sem1)

    def fill(g, par):
        pltpu.async_copy(emb_u_hbm.at[pl.ds(base_b + g * GB, GB)],
                         ugrp[par], sems[par])
        for q in range(ROWS_G // 120):
            pltpu.async_copy(
                v_emb_hbm.at[ctx_idx_v.at[pl.ds(g * ROWS_G + q * 120, 120)]],
                rows[par].at[pl.ds(q * 120, 120)], sems[par])

    def drain(par):
        pltpu.make_async_copy(emb_u_hbm.at[pl.ds(0, GB)],
                              ugrp[par], sems[par]).wait()
        for q in range(ROWS_G // 120):
            pltpu.make_async_copy(v_emb_hbm.at[pl.ds(0, 120)],
                                  rows[par].at[pl.ds(q * 120, 120)],
                                  sems[par]).wait()

    row_iota = lax.iota(jnp.int32, 16)
    cols = [jnp.full((16,), c, jnp.int32) for c in range(16)]

    fill(0, 0)
    fill(1, 1)

    @pl.loop(0, NG // 2)
    def _pair(t):
        for par in range(2):
            g = 2 * t + par
            drain(par)
            gbase = g * ROWS_G
            u_cache = {}
            for j in range(ROWS_G):
                bb, jj = divmod(j, NPAIR)
                if bb not in u_cache:
                    uc = [ugrp[par][bb, pl.ds(16 * k, 16)] for k in range(4)]
                    u_cache[bb] = (uc, [-c for c in uc])
                uc, nuc = u_cache[bb]
                ch = nuc if jj < P else uc
                part = rows[par][j, pl.ds(0, 16)] * ch[0]
                for k in range(1, 4):
                    part = part + rows[par][j, pl.ds(16 * k, 16)] * ch[k]
                tp_v[j % 16] = part
                if j % 16 == 15:
                    acc = plsc.load_gather(tp_v, [row_iota, cols[0]])
                    for c in range(1, 16):
                        acc = acc + plsc.load_gather(tp_v, [row_iota, cols[c]])
                    out_v[pl.ds(gbase + (j - 15), 16)] = acc

            @pl.when(g + 2 < NG)
            def _prefetch():
                fill(g + 2, par)

    pltpu.sync_copy(out_v, y_hbm.at[pl.ds(wid * OUT_W, OUT_W)])


def _tc_finish(y2d):
    nrows = y2d.shape[0]
    blk = 256
    grid = nrows // blk

    def body(y_ref, o_ref):
        i = pl.program_id(0)
        x = y_ref[...]
        sp = jnp.maximum(x, 0.0) + jnp.log1p(jnp.exp(-jnp.abs(x)))
        s = jnp.sum(sp) * (1.0 / B)

        @pl.when(i == 0)
        def _init():
            o_ref[...] = jnp.zeros_like(o_ref)

        o_ref[...] = o_ref[...] + s

    return pl.pallas_call(
        body,
        grid=(grid,),
        in_specs=[pl.BlockSpec((blk, 128), lambda i: (i, 0))],
        out_specs=pl.BlockSpec((1, 1), lambda i: (0, 0)),
        out_shape=jax.ShapeDtypeStruct((1, 1), jnp.float32),
    )(y2d)


def kernel(u, v, neg, wn, sim, not_sim, mismatch, u_emb, v_emb):
    del sim, not_sim, mismatch
    u_i = u.astype(jnp.int32)
    ctx = jnp.concatenate([v, neg, wn], axis=1).astype(jnp.int32).reshape(-1)
    emb_u = jnp.take(u_emb, u_i, axis=0)

    mesh = plsc.VectorSubcoreMesh(core_axis_name="c", subcore_axis_name="s")
    y = pl.kernel(
        _sc_body,
        out_type=jax.ShapeDtypeStruct((B * NPAIR,), jnp.float32),
        mesh=mesh,
        compiler_params=pltpu.CompilerParams(
            needs_layout_passes=False, use_tc_tiling_on_sc=False),
        scratch_types=[
            pltpu.VMEM((OUT_W,), jnp.int32),
            pltpu.VMEM((ROWS_G, DIM), jnp.float32),
            pltpu.VMEM((ROWS_G, DIM), jnp.float32),
            pltpu.VMEM((GB, DIM), jnp.float32),
            pltpu.VMEM((GB, DIM), jnp.float32),
            pltpu.VMEM((16, 16), jnp.float32),
            pltpu.VMEM((OUT_W,), jnp.float32),
            pltpu.SemaphoreType.DMA,
            pltpu.SemaphoreType.DMA,
        ],
    )(ctx, emb_u, v_emb)

    loss = _tc_finish(y.reshape(B * NPAIR // 128, 128))
    return loss[0, 0]
